# Initial kernel scaffold; baseline (speedup 1.0000x reference)
import jax
import jax.numpy as jnp
from jax.experimental import pallas as pl


def kernel(x, edge_index, W1, b1, W2, b2):
    raise NotImplementedError("write your pallas kernel here")


# speedup vs baseline: 22.2890x; 22.2890x over previous
import functools

import jax
import jax.numpy as jnp
from jax import lax
from jax.experimental import pallas as pl
from jax.experimental.pallas import tpu as pltpu
from jax.experimental.pallas import tpu_sc as plsc

N = 10000
D = 128
NC = 2
NS = 16
NW = NC * NS
L = 16
CHUNK = 128
NP = 10240
RPT = NP // NS
ZC = RPT // CHUNK

_mesh = plsc.VectorSubcoreMesh(core_axis_name="c", subcore_axis_name="s")


def _ceil_div(a, b):
    return (a + b - 1) // b


def _deg_kernel(ch):
    @functools.partial(
        pl.kernel,
        out_type=jax.ShapeDtypeStruct((NC, NP), jnp.float32),
        mesh=_mesh,
        scratch_types=[
            pltpu.VMEM_SHARED((NP,), jnp.float32),
            pltpu.VMEM((ch, CHUNK), jnp.int32),
            pltpu.VMEM((CHUNK,), jnp.float32),
            pltpu.VMEM((RPT,), jnp.float32),
        ],
    )
    def k(dstm_hbm, out_hbm, dacc, dst_v, ones_v, zero_v):
        cid = lax.axis_index("c")
        sid = lax.axis_index("s")
        wid = cid * NS + sid
        for j in range(CHUNK // L):
            ones_v[pl.ds(j * L, L)] = jnp.ones((L,), jnp.float32)

        def zfill(i, _):
            zero_v[pl.ds(i * L, L)] = jnp.zeros((L,), jnp.float32)
            return 0

        lax.fori_loop(0, RPT // L, zfill, 0)
        pltpu.sync_copy(zero_v, dacc.at[pl.ds(sid * RPT, RPT)])
        plsc.subcore_barrier()
        pltpu.sync_copy(dstm_hbm.at[wid], dst_v)

        def body(j, _):
            pltpu.sync_copy(ones_v, dacc.at[dst_v.at[j]], add=True)
            return 0

        lax.fori_loop(0, ch, body, 0)
        plsc.subcore_barrier()
        pltpu.sync_copy(dacc.at[pl.ds(sid * RPT, RPT)],
                        out_hbm.at[cid, pl.ds(sid * RPT, RPT)])

    return k


def _rows_kernel(ch):
    @functools.partial(
        pl.kernel,
        out_type=jax.ShapeDtypeStruct((NC, NP, D), jnp.float32),
        mesh=_mesh,
        scratch_types=[
            pltpu.VMEM_SHARED((NP, D), jnp.float32),
            pltpu.VMEM((ch, CHUNK), jnp.int32),
            pltpu.VMEM((ch, CHUNK), jnp.int32),
            pltpu.VMEM((CHUNK, D), jnp.float32),
            pltpu.SemaphoreType.DMA,
        ],
    )
    def k(table_hbm, srcm_hbm, dstm_hbm, out_hbm, acc, src_v, dst_v, rows_v, sem):
        cid = lax.axis_index("c")
        sid = lax.axis_index("s")
        wid = cid * NS + sid

        def zfill(i, _):
            for j in range(D // L):
                rows_v[i, pl.ds(j * L, L)] = jnp.zeros((L,), jnp.float32)
            return 0

        lax.fori_loop(0, CHUNK, zfill, 0)
        for j in range(ZC):
            pltpu.sync_copy(
                rows_v, acc.at[pl.ds(sid * RPT + j * CHUNK, CHUNK)])
        plsc.subcore_barrier()
        pltpu.sync_copy(srcm_hbm.at[wid], src_v)
        pltpu.sync_copy(dstm_hbm.at[wid], dst_v)

        def body(j, _):
            pltpu.async_copy(table_hbm.at[src_v.at[j]], rows_v, sem).wait()
            pltpu.sync_copy(rows_v, acc.at[dst_v.at[j]], add=True)
            return 0

        lax.fori_loop(0, ch, body, 0)
        plsc.subcore_barrier()
        pltpu.sync_copy(acc.at[pl.ds(sid * RPT, RPT)],
                        out_hbm.at[cid, pl.ds(sid * RPT, RPT)])

    return k


_R = 1000


def _tc1(x, W1, dga, dgb):
    def body(x_b, w_b, da_b, db_b, dis_b, t1_b):
        dis = lax.rsqrt(da_b[...] + db_b[...] + 1.0)
        dis_b[...] = dis
        t1_b[...] = jnp.dot(
            x_b[...], w_b[...], preferred_element_type=jnp.float32) * dis

    return pl.pallas_call(
        body,
        grid=(N // _R,),
        in_specs=[
            pl.BlockSpec((_R, D), lambda i: (i, 0)),
            pl.BlockSpec((D, D), lambda i: (0, 0)),
            pl.BlockSpec((_R, 1), lambda i: (i, 0)),
            pl.BlockSpec((_R, 1), lambda i: (i, 0)),
        ],
        out_specs=[
            pl.BlockSpec((_R, 1), lambda i: (i, 0)),
            pl.BlockSpec((_R, D), lambda i: (i, 0)),
        ],
        out_shape=[
            jax.ShapeDtypeStruct((N, 1), jnp.float32),
            jax.ShapeDtypeStruct((N, D), jnp.float32),
        ],
    )(x, W1, dga, dgb)


def _tc2(t1, p0, p1, dis, b1, W2):
    def body(t_b, p0_b, p1_b, d_b, b_b, w_b, o_b):
        z = d_b[...] * (p0_b[...] + p1_b[...] + t_b[...]) + b_b[...]
        z = jnp.maximum(z, 0.0)
        o_b[...] = jnp.dot(
            z, w_b[...], preferred_element_type=jnp.float32) * d_b[...]

    return pl.pallas_call(
        body,
        grid=(N // _R,),
        in_specs=[
            pl.BlockSpec((_R, D), lambda i: (i, 0)),
            pl.BlockSpec((_R, D), lambda i: (i, 0)),
            pl.BlockSpec((_R, D), lambda i: (i, 0)),
            pl.BlockSpec((_R, 1), lambda i: (i, 0)),
            pl.BlockSpec((1, D), lambda i: (0, 0)),
            pl.BlockSpec((D, D), lambda i: (0, 0)),
        ],
        out_specs=pl.BlockSpec((_R, D), lambda i: (i, 0)),
        out_shape=jax.ShapeDtypeStruct((N, D), jnp.float32),
    )(t1, p0, p1, dis, b1, W2)


def _tc3(t2, q0, q1, dis, b2):
    def body(t_b, q0_b, q1_b, d_b, b_b, o_b):
        o_b[...] = d_b[...] * (q0_b[...] + q1_b[...] + t_b[...]) + b_b[...]

    return pl.pallas_call(
        body,
        grid=(N // _R,),
        in_specs=[
            pl.BlockSpec((_R, D), lambda i: (i, 0)),
            pl.BlockSpec((_R, D), lambda i: (i, 0)),
            pl.BlockSpec((_R, D), lambda i: (i, 0)),
            pl.BlockSpec((_R, 1), lambda i: (i, 0)),
            pl.BlockSpec((1, D), lambda i: (0, 0)),
        ],
        out_specs=pl.BlockSpec((_R, D), lambda i: (i, 0)),
        out_shape=jax.ShapeDtypeStruct((N, D), jnp.float32),
    )(t2, q0, q1, dis, b2)


def kernel(x, edge_index, W1, b1, W2, b2):
    E = edge_index.shape[1]
    ch = _ceil_div(E, NW * CHUNK)
    e_pad = NW * ch * CHUNK
    pad = e_pad - E
    src = edge_index[0]
    dst = edge_index[1]
    if pad:
        ar = jnp.arange(pad, dtype=jnp.int32)
        src = jnp.concatenate([src, (ar * 997) % N])
        dst = jnp.concatenate([dst, N + (ar % (NP - N))])
    srcm = src.reshape(NW, ch, CHUNK)
    dstm = dst.reshape(NW, ch, CHUNK)

    degp = _deg_kernel(ch)(dstm)
    dga = degp[0, :N, None]
    dgb = degp[1, :N, None]
    dis, t1 = _tc1(x, W1, dga, dgb)

    rows = _rows_kernel(ch)
    acc1 = rows(t1, srcm, dstm)
    t2 = _tc2(t1, acc1[0, :N], acc1[1, :N], dis, b1.reshape(1, D), W2)
    acc2 = rows(t2, srcm, dstm)
    return _tc3(t2, acc2[0, :N], acc2[1, :N], dis, b2.reshape(1, D))
